# baseline (device time: 65111 ns/iter reference)
import jax
import jax.numpy as jnp
from jax import lax
from jax.experimental import pallas as pl
from jax.experimental.pallas import tpu as pltpu

N_DEV = 4
N_PEER = N_DEV - 1
N_LAYER = 3


def kernel(x, Win0, Wout0, Win1, Wout1, Win2, Wout2):
    m_per, d = x.shape
    M = N_DEV * m_per

    def body(x_ref, win0_ref, wout0_ref, win1_ref, wout1_ref, win2_ref,
             wout2_ref, out_ref,
             X_ref, psend_ref, rs_ref,
             ag_ssem, ag_rsem, rs_ssem, rs_rsem):
        my = lax.axis_index("i")

        barrier_sem = pltpu.get_barrier_semaphore()
        for j in range(1, N_DEV):
            pl.semaphore_signal(
                barrier_sem, inc=1,
                device_id=((my + j) % N_DEV,),
                device_id_type=pl.DeviceIdType.MESH,
            )
        pl.semaphore_wait(barrier_sem, N_PEER)

        def ag_send_all(phase):
            sends = []
            for j in range(1, N_DEV):
                s = pltpu.make_async_remote_copy(
                    src_ref=X_ref.at[pl.ds(my * m_per, m_per), :],
                    dst_ref=X_ref.at[pl.ds(my * m_per, m_per), :],
                    send_sem=ag_ssem.at[phase, j - 1],
                    recv_sem=ag_rsem.at[phase, j - 1],
                    device_id=((my + j) % N_DEV,),
                    device_id_type=pl.DeviceIdType.MESH,
                )
                s.start()
                sends.append(s)
            return sends

        def ag_wait_recv(phase, j):
            rows = (my + j) % N_DEV
            recv = pltpu.make_async_remote_copy(
                src_ref=X_ref.at[pl.ds(rows * m_per, m_per), :],
                dst_ref=X_ref.at[pl.ds(rows * m_per, m_per), :],
                send_sem=ag_ssem.at[phase, j - 1],
                recv_sem=ag_rsem.at[phase, (N_DEV - j) - 1],
                device_id=(0,),
                device_id_type=pl.DeviceIdType.MESH,
            )
            recv.wait_recv()

        X_ref[pl.ds(my * m_per, m_per), :] = x_ref[...].astype(jnp.bfloat16)
        ag_sends = ag_send_all(0)

        for l, (win_ref, wout_ref) in enumerate(
            ((win0_ref, wout0_ref), (win1_ref, wout1_ref), (win2_ref, wout2_ref))
        ):
            win = win_ref[...].astype(jnp.bfloat16)
            wout = wout_ref[...].astype(jnp.bfloat16)

            def compute_chunk(owner):
                Xc = X_ref[pl.ds(owner * m_per, m_per), :]
                h = jnp.dot(Xc, win, preferred_element_type=jnp.float32)
                h = jnp.maximum(h, 0.0).astype(jnp.bfloat16)
                return jnp.dot(h, wout, preferred_element_type=jnp.float32)

            acc = compute_chunk(my).astype(jnp.bfloat16).astype(jnp.float32)

            rs_sends = []
            for j in range(1, N_DEV):
                ag_wait_recv(l, j)
                owner = (my + j) % N_DEV
                psend_ref[j - 1] = compute_chunk(owner).astype(jnp.bfloat16)
                s = pltpu.make_async_remote_copy(
                    src_ref=psend_ref.at[j - 1],
                    dst_ref=rs_ref.at[l, j - 1],
                    send_sem=rs_ssem.at[l, j - 1],
                    recv_sem=rs_rsem.at[l, j - 1],
                    device_id=(owner,),
                    device_id_type=pl.DeviceIdType.MESH,
                )
                s.start()
                rs_sends.append(s)

            for j in range(1, N_DEV):
                slot = (N_DEV - j) - 1
                recv = pltpu.make_async_remote_copy(
                    src_ref=rs_ref.at[l, slot],
                    dst_ref=rs_ref.at[l, slot],
                    send_sem=rs_ssem.at[l, slot],
                    recv_sem=rs_rsem.at[l, slot],
                    device_id=(0,),
                    device_id_type=pl.DeviceIdType.MESH,
                )
                recv.wait_recv()
                acc = acc + rs_ref[l, slot].astype(jnp.float32)

            for s in ag_sends:
                s.wait_send()
            X_ref[pl.ds(my * m_per, m_per), :] = acc.astype(jnp.bfloat16)
            ag_sends = ag_send_all(l + 1)
            for s in rs_sends:
                s.wait_send()

        for j in range(1, N_DEV):
            ag_wait_recv(N_LAYER, j)
        for s in ag_sends:
            s.wait_send()
        out_ref[...] = X_ref[...].astype(jnp.float32)

    return pl.pallas_call(
        body,
        out_shape=jax.ShapeDtypeStruct((M, d), jnp.float32),
        in_specs=[pl.BlockSpec(memory_space=pltpu.VMEM)] * 7,
        out_specs=pl.BlockSpec(memory_space=pltpu.VMEM),
        scratch_shapes=[
            pltpu.VMEM((M, d), jnp.bfloat16),
            pltpu.VMEM((N_PEER, m_per, d), jnp.bfloat16),
            pltpu.VMEM((N_LAYER, N_PEER, m_per, d), jnp.bfloat16),
            pltpu.SemaphoreType.DMA((N_LAYER + 1, N_PEER)),
            pltpu.SemaphoreType.DMA((N_LAYER + 1, N_PEER)),
            pltpu.SemaphoreType.DMA((N_LAYER, N_PEER)),
            pltpu.SemaphoreType.DMA((N_LAYER, N_PEER)),
        ],
        compiler_params=pltpu.CompilerParams(
            collective_id=0,
            vmem_limit_bytes=100 * 1024 * 1024,
        ),
    )(x, Win0, Wout0, Win1, Wout1, Win2, Wout2)


# device time: 52887 ns/iter; 1.2311x vs baseline; 1.2311x over previous
import os

import jax
import jax.numpy as jnp
from jax import lax
from jax.experimental import pallas as pl
from jax.experimental.pallas import tpu as pltpu

_VARIANT = os.environ.get("KERNEL_VARIANT", "full")
_DO_COMM = _VARIANT != "compute"
_DO_COMPUTE = _VARIANT != "comm"

N_DEV = 4
N_PEER = N_DEV - 1
N_LAYER = 3


def kernel(x, Win0, Wout0, Win1, Wout1, Win2, Wout2):
    m_per, d = x.shape
    h_per = Win0.shape[1]
    M = N_DEV * m_per

    def body(x_ref, win0_ref, wout0_ref, win1_ref, wout1_ref, win2_ref,
             wout2_ref, out_ref,
             X_ref, psend_ref, rs_ref, winv_ref, woutv_ref,
             ag_ssem, ag_rsem, rs_ssem, rs_rsem, wdma_sem):
        my = lax.axis_index("i")

        wdmas = []
        for l, (src, dst) in enumerate((
            (win0_ref, winv_ref.at[0]), (wout0_ref, woutv_ref.at[0]),
            (win1_ref, winv_ref.at[1]), (wout1_ref, woutv_ref.at[1]),
            (win2_ref, winv_ref.at[2]), (wout2_ref, woutv_ref.at[2]),
        )):
            c = pltpu.make_async_copy(src, dst, wdma_sem.at[l])
            c.start()
            wdmas.append(c)

        if _DO_COMM:
            barrier_sem = pltpu.get_barrier_semaphore()
            for j in range(1, N_DEV):
                pl.semaphore_signal(
                    barrier_sem, inc=1,
                    device_id=((my + j) % N_DEV,),
                    device_id_type=pl.DeviceIdType.MESH,
                )
            pl.semaphore_wait(barrier_sem, N_PEER)

        def ag_send_all(phase):
            sends = []
            if not _DO_COMM:
                return sends
            for j in range(1, N_DEV):
                s = pltpu.make_async_remote_copy(
                    src_ref=X_ref.at[pl.ds(my * m_per, m_per), :],
                    dst_ref=X_ref.at[pl.ds(my * m_per, m_per), :],
                    send_sem=ag_ssem.at[phase, j - 1],
                    recv_sem=ag_rsem.at[phase, j - 1],
                    device_id=((my + j) % N_DEV,),
                    device_id_type=pl.DeviceIdType.MESH,
                )
                s.start()
                sends.append(s)
            return sends

        def ag_wait_recv(phase, j):
            if not _DO_COMM:
                return
            rows = (my + j) % N_DEV
            recv = pltpu.make_async_remote_copy(
                src_ref=X_ref.at[pl.ds(rows * m_per, m_per), :],
                dst_ref=X_ref.at[pl.ds(rows * m_per, m_per), :],
                send_sem=ag_ssem.at[phase, j - 1],
                recv_sem=ag_rsem.at[phase, (N_DEV - j) - 1],
                device_id=(0,),
                device_id_type=pl.DeviceIdType.MESH,
            )
            recv.wait_recv()

        X_ref[pl.ds(my * m_per, m_per), :] = x_ref[...].astype(jnp.bfloat16)
        ag_sends = ag_send_all(0)

        for l in range(N_LAYER):
            wdmas[2 * l].wait()
            win = winv_ref[l].astype(jnp.bfloat16)
            wdmas[2 * l + 1].wait()
            wout = woutv_ref[l].astype(jnp.bfloat16)

            def compute_chunk(owner):
                Xc = X_ref[pl.ds(owner * m_per, m_per), :]
                if not _DO_COMPUTE:
                    return Xc.astype(jnp.float32)
                h = jnp.dot(Xc, win, preferred_element_type=jnp.float32)
                h = jnp.maximum(h, 0.0).astype(jnp.bfloat16)
                return jnp.dot(h, wout, preferred_element_type=jnp.float32)

            acc = compute_chunk(my).astype(jnp.bfloat16).astype(jnp.float32)

            rs_sends = []
            for j in range(1, N_DEV):
                ag_wait_recv(l, j)
                owner = (my + j) % N_DEV
                psend_ref[j - 1] = compute_chunk(owner).astype(jnp.bfloat16)
                if _DO_COMM:
                    s = pltpu.make_async_remote_copy(
                        src_ref=psend_ref.at[j - 1],
                        dst_ref=rs_ref.at[l, j - 1],
                        send_sem=rs_ssem.at[l, j - 1],
                        recv_sem=rs_rsem.at[l, j - 1],
                        device_id=(owner,),
                        device_id_type=pl.DeviceIdType.MESH,
                    )
                    s.start()
                    rs_sends.append(s)

            for j in range(1, N_DEV) if _DO_COMM else ():
                slot = (N_DEV - j) - 1
                recv = pltpu.make_async_remote_copy(
                    src_ref=rs_ref.at[l, slot],
                    dst_ref=rs_ref.at[l, slot],
                    send_sem=rs_ssem.at[l, slot],
                    recv_sem=rs_rsem.at[l, slot],
                    device_id=(0,),
                    device_id_type=pl.DeviceIdType.MESH,
                )
                recv.wait_recv()
                acc = acc + rs_ref[l, slot].astype(jnp.float32)

            for s in ag_sends:
                s.wait_send()
            X_ref[pl.ds(my * m_per, m_per), :] = acc.astype(jnp.bfloat16)
            ag_sends = ag_send_all(l + 1)
            for s in rs_sends:
                s.wait_send()

        for j in range(1, N_DEV):
            ag_wait_recv(N_LAYER, j)
        for s in ag_sends:
            s.wait_send()
        out_ref[...] = X_ref[...].astype(jnp.float32)

    return pl.pallas_call(
        body,
        out_shape=jax.ShapeDtypeStruct((M, d), jnp.float32),
        in_specs=[pl.BlockSpec(memory_space=pltpu.VMEM)]
        + [pl.BlockSpec(memory_space=pl.ANY)] * 6,
        out_specs=pl.BlockSpec(memory_space=pltpu.VMEM),
        scratch_shapes=[
            pltpu.VMEM((M, d), jnp.bfloat16),
            pltpu.VMEM((N_PEER, m_per, d), jnp.bfloat16),
            pltpu.VMEM((N_LAYER, N_PEER, m_per, d), jnp.bfloat16),
            pltpu.VMEM((N_LAYER, d, h_per), jnp.float32),
            pltpu.VMEM((N_LAYER, h_per, d), jnp.float32),
            pltpu.SemaphoreType.DMA((N_LAYER + 1, N_PEER)),
            pltpu.SemaphoreType.DMA((N_LAYER + 1, N_PEER)),
            pltpu.SemaphoreType.DMA((N_LAYER, N_PEER)),
            pltpu.SemaphoreType.DMA((N_LAYER, N_PEER)),
            pltpu.SemaphoreType.DMA((2 * N_LAYER,)),
        ],
        compiler_params=pltpu.CompilerParams(
            collective_id=0 if _DO_COMM else None,
            vmem_limit_bytes=100 * 1024 * 1024,
        ),
    )(x, Win0, Wout0, Win1, Wout1, Win2, Wout2)
